# baseline (speedup 1.0000x reference)
import functools

import jax
import jax.numpy as jnp
from jax import lax
from jax.experimental import pallas as pl
from jax.experimental.pallas import tpu as pltpu
from jax.experimental.pallas import tpu_sc as plsc

NC = 2
NS = 16
NW = NC * NS
C = 128
NCHUNK = 80


def _linear_body(x_ref, wt_ref, b_ref, o_ref):
    o_ref[...] = (
        jnp.dot(x_ref[...], wt_ref[...], preferred_element_type=jnp.float32)
        + b_ref[...]
    )


def _combine_body(p_ref, norm_ref, o_ref):
    n = o_ref.shape[0]
    o_ref[...] = norm_ref[...] * (p_ref[0, :n] + p_ref[1, :n])


@functools.lru_cache(maxsize=None)
def _make_agg(N, D):
    NP = ((N + 8 * NS) // (8 * NS)) * (8 * NS)
    ROWS = NP // NS
    n_pairs = NCHUNK // 2

    mesh = plsc.VectorSubcoreMesh(
        core_axis_name="c", subcore_axis_name="s", num_cores=NC, num_subcores=NS
    )

    @functools.partial(
        pl.kernel,
        out_type=jax.ShapeDtypeStruct((NC, NP, D), jnp.float32),
        mesh=mesh,
        scratch_types=[
            pltpu.VMEM((2, 2, C), jnp.int32),
            pltpu.VMEM((2, C, D), jnp.float32),
            pltpu.VMEM_SHARED((NP, D), jnp.float32),
            pltpu.SemaphoreType.DMA,
            pltpu.SemaphoreType.DMA,
            pltpu.SemaphoreType.DMA,
            pltpu.SemaphoreType.DMA,
        ],
    )
    def agg(h_hbm, edges_hbm, zeros_hbm, out_hbm,
            eb_v, rows_v, acc_sh, semg0, semg1, semi0, semi1):
        cid = lax.axis_index("c")
        sid = lax.axis_index("s")
        wid = sid * NC + cid
        semg = (semg0, semg1)
        semi = (semi0, semi1)
        pltpu.sync_copy(zeros_hbm, acc_sh.at[pl.ds(sid * ROWS, ROWS)])
        plsc.subcore_barrier()

        pltpu.sync_copy(edges_hbm.at[wid, 0], eb_v.at[0])
        pltpu.async_copy(edges_hbm.at[wid, 1], eb_v.at[1], semi[1])
        pltpu.async_copy(h_hbm.at[eb_v.at[0, 0]], rows_v.at[0], semg[0])

        def pair(g, carry):
            for b in range(2):
                i = g * 2 + b
                pltpu.make_async_copy(
                    h_hbm.at[eb_v.at[b, 0]], rows_v.at[b], semg[b]
                ).wait()
                pltpu.sync_copy(rows_v.at[b], acc_sh.at[pl.ds(0, C)])
                if b == 0:
                    @pl.when(g < n_pairs - 1)
                    def _():
                        pltpu.async_copy(
                            edges_hbm.at[wid, i + 2], eb_v.at[b], semi[b]
                        )
                        pltpu.make_async_copy(
                            edges_hbm.at[wid, i + 1], eb_v.at[1 - b], semi[1 - b]
                        ).wait()
                        pltpu.async_copy(
                            h_hbm.at[eb_v.at[1 - b, 0]], rows_v.at[1 - b],
                            semg[1 - b],
                        )

                    @pl.when(g == n_pairs - 1)
                    def _():
                        pltpu.make_async_copy(
                            edges_hbm.at[wid, i + 1], eb_v.at[1 - b], semi[1 - b]
                        ).wait()
                        pltpu.async_copy(
                            h_hbm.at[eb_v.at[1 - b, 0]], rows_v.at[1 - b],
                            semg[1 - b],
                        )
                else:
                    @pl.when(g < n_pairs - 1)
                    def _():
                        pltpu.async_copy(
                            edges_hbm.at[wid, i + 2], eb_v.at[b], semi[b]
                        )
                        pltpu.make_async_copy(
                            edges_hbm.at[wid, i + 1], eb_v.at[1 - b], semi[1 - b]
                        ).wait()
                        pltpu.async_copy(
                            h_hbm.at[eb_v.at[1 - b, 0]], rows_v.at[1 - b],
                            semg[1 - b],
                        )
            return carry

        lax.fori_loop(0, n_pairs, pair, 0)
        plsc.subcore_barrier()
        pltpu.sync_copy(
            acc_sh.at[pl.ds(sid * ROWS, ROWS)],
            out_hbm.at[cid, pl.ds(sid * ROWS, ROWS)],
        )

    return agg


def kernel(x, edge_index, norm, W, b):
    N, D_in = x.shape
    D_out = W.shape[0]
    E = edge_index.shape[1]

    h = pl.pallas_call(
        _linear_body,
        out_shape=jax.ShapeDtypeStruct((N, D_out), jnp.float32),
    )(x, W.T, b.reshape(1, D_out))

    E_pad = NW * NCHUNK * C
    src = jnp.concatenate(
        [edge_index[0], jnp.zeros((E_pad - E,), jnp.int32)]
    ).reshape(NW, NCHUNK, 1, C)
    dst = jnp.concatenate(
        [edge_index[1], jnp.full((E_pad - E,), N, jnp.int32)]
    ).reshape(NW, NCHUNK, 1, C)
    edges = jnp.concatenate([src, dst], axis=2)
    NP = ((N + 8 * NS) // (8 * NS)) * (8 * NS)
    zeros = jnp.zeros((NP // NS, D_out), dtype=jnp.float32)
    partials = _make_agg(N, D_out)(h, edges, zeros)

    out = pl.pallas_call(
        _combine_body,
        out_shape=jax.ShapeDtypeStruct((N, D_out), jnp.float32),
    )(partials, norm)
    return out


# speedup vs baseline: 1.1646x; 1.1646x over previous
import functools

import jax
import jax.numpy as jnp
from jax import lax
from jax.experimental import pallas as pl
from jax.experimental.pallas import tpu as pltpu
from jax.experimental.pallas import tpu_sc as plsc

NC = 2
NS = 16
NW = NC * NS
C = 128
NCHUNK = 80
NI = 4
NG = 2
QUAD = 4


def _linear_body(x_ref, wt_ref, b_ref, o_ref):
    o_ref[...] = (
        jnp.dot(x_ref[...], wt_ref[...], preferred_element_type=jnp.float32)
        + b_ref[...]
    )


def _combine_body(p_ref, norm_ref, o_ref):
    n = o_ref.shape[0]
    o_ref[...] = norm_ref[...] * (p_ref[0, :n] + p_ref[1, :n])


@functools.lru_cache(maxsize=None)
def _make_agg(N, D):
    NP = ((N + 8 * NS) // (8 * NS)) * (8 * NS)
    ROWS = NP // NS
    n_quads = NCHUNK // QUAD

    mesh = plsc.VectorSubcoreMesh(
        core_axis_name="c", subcore_axis_name="s", num_cores=NC, num_subcores=NS
    )

    @functools.partial(
        pl.kernel,
        out_type=jax.ShapeDtypeStruct((NC, NP, D), jnp.float32),
        mesh=mesh,
        scratch_types=[
            pltpu.VMEM((NI, 2, C), jnp.int32),
            pltpu.VMEM((NG, C, D), jnp.float32),
            pltpu.VMEM_SHARED((NP, D), jnp.float32),
            [pltpu.SemaphoreType.DMA] * NI,
            [pltpu.SemaphoreType.DMA] * NG,
        ],
    )
    def agg(h_hbm, edges_hbm, zeros_hbm, out_hbm,
            eb_v, rows_v, acc_sh, semi, semg):
        cid = lax.axis_index("c")
        sid = lax.axis_index("s")
        wid = sid * NC + cid
        pltpu.sync_copy(zeros_hbm, acc_sh.at[pl.ds(sid * ROWS, ROWS)])
        plsc.subcore_barrier()

        def idx_load(i, slot):
            pltpu.async_copy(edges_hbm.at[wid, i], eb_v.at[slot], semi[slot])

        def idx_wait(i, slot):
            pltpu.make_async_copy(
                edges_hbm.at[wid, i], eb_v.at[slot], semi[slot]
            ).wait()

        def gather_start(islot, rslot):
            pltpu.async_copy(
                h_hbm.at[eb_v.at[islot, 0]], rows_v.at[rslot], semg[rslot]
            )

        def gather_wait(islot, rslot):
            pltpu.make_async_copy(
                h_hbm.at[eb_v.at[islot, 0]], rows_v.at[rslot], semg[rslot]
            ).wait()

        idx_load(0, 0)
        idx_load(1, 1)
        idx_load(2, 2)
        idx_wait(0, 0)
        gather_start(0, 0)

        def quad(g, carry):
            for q in range(QUAD):
                i = g * QUAD + q
                ib, ib1 = q % NI, (q + 1) % NI
                rb, rb1 = q % NG, (q + 1) % NG
                last = n_quads - 1

                def step_ace():
                    idx_wait(i + 1, ib1)
                    gather_start(ib1, rb1)

                if q == QUAD - 1:
                    @pl.when(g < last)
                    def _():
                        step_ace()
                else:
                    step_ace()

                gather_wait(ib, rb)
                pltpu.sync_copy(rows_v.at[rb], acc_sh.at[eb_v.at[ib, 1]],
                                add=True)

                if q == 0:
                    idx_load(i + 3, (q + 3) % NI)
                else:
                    @pl.when(g < last)
                    def _():
                        idx_load(i + 3, (q + 3) % NI)
            return carry

        lax.fori_loop(0, n_quads, quad, 0)
        plsc.subcore_barrier()
        pltpu.sync_copy(
            acc_sh.at[pl.ds(sid * ROWS, ROWS)],
            out_hbm.at[cid, pl.ds(sid * ROWS, ROWS)],
        )

    return agg


def kernel(x, edge_index, norm, W, b):
    N, D_in = x.shape
    D_out = W.shape[0]
    E = edge_index.shape[1]

    h = pl.pallas_call(
        _linear_body,
        out_shape=jax.ShapeDtypeStruct((N, D_out), jnp.float32),
    )(x, W.T, b.reshape(1, D_out))

    E_pad = NW * NCHUNK * C
    src = jnp.concatenate(
        [edge_index[0], jnp.zeros((E_pad - E,), jnp.int32)]
    ).reshape(NW, NCHUNK, 1, C)
    dst = jnp.concatenate(
        [edge_index[1], jnp.full((E_pad - E,), N, jnp.int32)]
    ).reshape(NW, NCHUNK, 1, C)
    edges = jnp.concatenate([src, dst], axis=2)
    NP = ((N + 8 * NS) // (8 * NS)) * (8 * NS)
    zeros = jnp.zeros((NP // NS, D_out), dtype=jnp.float32)
    partials = _make_agg(N, D_out)(h, edges, zeros)

    out = pl.pallas_call(
        _combine_body,
        out_shape=jax.ShapeDtypeStruct((N, D_out), jnp.float32),
    )(partials, norm)
    return out
